# Initial kernel scaffold; baseline (speedup 1.0000x reference)
import jax
import jax.numpy as jnp
from jax.experimental import pallas as pl


def kernel(p_x, p_edge_index, pos_N, pos_CA, pos_C, pos_CB, Wg1, bg1, Wg2, bg2, ln1g, ln1b, ln2g, ln2b, Wq1, Wk1, Wv1, Wo1, bo1, ag1, ab1, Wq2, Wk2, Wv2, Wo2, bo2, ag2, ab2, Wf1, bf1, Wf2, bf2):
    raise NotImplementedError("write your pallas kernel here")


# speedup vs baseline: 10.1527x; 10.1527x over previous
import functools

import jax
import jax.numpy as jnp
from jax import lax
from jax.experimental import pallas as pl
from jax.experimental.pallas import tpu as pltpu
from jax.experimental.pallas import tpu_sc as plsc

_B, _L, _H, _QK, _VD = 64, 256, 4, 16, 16
_N = _B * _L
_E = 524288
_DIN, _DH = 33, 128
_OD = _H * _VD + _H * 7

_NC, _NS = 2, 16
_CH = 128
_ROWS = _E // _CH
_NLOC = _N // _NS

_f32 = jnp.float32


@functools.lru_cache(maxsize=None)
def _sc_mesh():
    return plsc.VectorSubcoreMesh(core_axis_name="c", subcore_axis_name="s",
                                  num_cores=_NC, num_subcores=_NS)


def _deg_body(dst_r, out, idx_v, zbuf, obuf, acc):
    c = lax.axis_index("c")
    s = lax.axis_index("s")
    w = c * _NS + s
    rpw = _ROWS // (_NC * _NS)
    pltpu.sync_copy(dst_r.at[pl.ds(w * rpw, rpw)], idx_v)

    def fill(i, _):
        zbuf[i] = jnp.zeros((16,), _f32)
        obuf[i] = jnp.full((16,), 1.0, _f32)
        return 0

    lax.fori_loop(0, _CH, fill, 0)
    for kk in range(_NLOC // _CH):
        pltpu.sync_copy(zbuf, acc.at[pl.ds(s * _NLOC + kk * _CH, _CH)])
    plsc.subcore_barrier()

    def step(j, _):
        pltpu.sync_copy(obuf, acc.at[idx_v.at[j]], add=True)
        return 0

    lax.fori_loop(0, rpw, step, 0)
    plsc.subcore_barrier()
    pltpu.sync_copy(acc.at[pl.ds(s * _NLOC, _NLOC)],
                    out.at[pl.ds(c * _N + s * _NLOC, _NLOC)])


@functools.lru_cache(maxsize=None)
def _deg_kernel():
    return pl.kernel(
        _deg_body,
        out_type=jax.ShapeDtypeStruct((_NC * _N, 16), _f32),
        mesh=_sc_mesh(),
        scratch_types=[
            pltpu.VMEM((_ROWS // (_NC * _NS), _CH), jnp.int32),
            pltpu.VMEM((_CH, 16), _f32),
            pltpu.VMEM((_CH, 16), _f32),
            pltpu.VMEM_SHARED((_N, 16), _f32),
        ],
        compiler_params=pltpu.CompilerParams(use_tc_tiling_on_sc=False),
    )


def _deg_call(dst_r):
    return _deg_kernel()(dst_r)


def _scat_body(y4, src_q, dst_r, out, isrc, idst, buf, acc, sem):
    c = lax.axis_index("c")
    s = lax.axis_index("s")
    rps = _ROWS // _NS
    pltpu.sync_copy(dst_r.at[pl.ds(s * rps, rps)], idst)
    for j in range(2):
        q = 2 * c + j
        pltpu.sync_copy(src_q.at[q, pl.ds(s * rps, rps)], isrc)
        pltpu.sync_copy(y4.at[pl.ds(q * _N + s * _NLOC, _NLOC)],
                        acc.at[pl.ds(s * _NLOC, _NLOC)])
        plsc.subcore_barrier()

        def step(i, _):
            pltpu.async_copy(y4.at[isrc.at[i]], buf, sem).wait()
            pltpu.sync_copy(buf, acc.at[idst.at[i]], add=True)
            return 0

        lax.fori_loop(0, rps, step, 0)
        plsc.subcore_barrier()
        pltpu.sync_copy(acc.at[pl.ds(s * _NLOC, _NLOC)],
                        out.at[pl.ds(q * _N + s * _NLOC, _NLOC)])


@functools.lru_cache(maxsize=None)
def _scat_kernel():
    return pl.kernel(
        _scat_body,
        out_type=jax.ShapeDtypeStruct((4 * _N, 32), _f32),
        mesh=_sc_mesh(),
        scratch_types=[
            pltpu.VMEM((_ROWS // _NS, _CH), jnp.int32),
            pltpu.VMEM((_ROWS // _NS, _CH), jnp.int32),
            pltpu.VMEM((_CH, 32), _f32),
            pltpu.VMEM_SHARED((_N, 32), _f32),
            pltpu.SemaphoreType.DMA,
        ],
        compiler_params=pltpu.CompilerParams(use_tc_tiling_on_sc=False),
    )


def _scat_call(y4, src_q, dst_r):
    return _scat_kernel()(y4, src_q, dst_r)


def _ln(x, g, b):
    m = jnp.mean(x, axis=-1, keepdims=True)
    v = jnp.mean((x - m) * (x - m), axis=-1, keepdims=True)
    return (x - m) / jnp.sqrt(v + 1e-5) * g + b


def _prep_body(px_r, pn_r, pca_r, pc_r, pcb_r, dp_r, wg1_r,
               y_r, geom_r, dinv_r):
    deg = dp_r[0, :, 0:1] + dp_r[1, :, 0:1] + 1.0
    dinv = lax.rsqrt(deg)
    dinv_r[...] = dinv
    h = jnp.dot(px_r[...], wg1_r[...], preferred_element_type=_f32)
    y = h * dinv
    for q in range(4):
        y_r[q] = y[:, 32 * q:32 * q + 32]
    pn, pca, pc, pcb = pn_r[...], pca_r[...], pc_r[...], pcb_r[...]
    e0 = pn - pca
    e1 = pc - pca
    e0 = e0 / jnp.sqrt(jnp.sum(e0 * e0, axis=1, keepdims=True) + 1e-10)
    dot = jnp.sum(e0 * e1, axis=1, keepdims=True)
    e1 = e1 - e0 * dot
    e1 = e1 / jnp.sqrt(jnp.sum(e1 * e1, axis=1, keepdims=True) + 1e-10)
    a0, a1, a2 = e0[:, 0:1], e0[:, 1:2], e0[:, 2:3]
    b0, b1, b2 = e1[:, 0:1], e1[:, 1:2], e1[:, 2:3]
    e2 = jnp.concatenate(
        [a1 * b2 - a2 * b1, a2 * b0 - a0 * b2, a0 * b1 - a1 * b0], axis=1)
    zero = jnp.zeros((pn.shape[0], 1), _f32)
    geom_r[...] = jnp.concatenate([pcb, pca, e0, e1, e2, zero], axis=1)


def _prep_call(p_x, pos_n, pos_ca, pos_c, pos_cb, deg_parts, wg1):
    nb = 16
    blk = _N // nb
    return pl.pallas_call(
        _prep_body,
        grid=(nb,),
        in_specs=[
            pl.BlockSpec((blk, _DIN), lambda i: (i, 0)),
            pl.BlockSpec((blk, 3), lambda i: (i, 0)),
            pl.BlockSpec((blk, 3), lambda i: (i, 0)),
            pl.BlockSpec((blk, 3), lambda i: (i, 0)),
            pl.BlockSpec((blk, 3), lambda i: (i, 0)),
            pl.BlockSpec((2, blk, 16), lambda i: (0, i, 0)),
            pl.BlockSpec((_DIN, _DH), lambda i: (0, 0)),
        ],
        out_specs=[
            pl.BlockSpec((4, blk, 32), lambda i: (0, i, 0)),
            pl.BlockSpec((blk, 16), lambda i: (i, 0)),
            pl.BlockSpec((blk, 1), lambda i: (i, 0)),
        ],
        out_shape=[
            jax.ShapeDtypeStruct((4, _N, 32), _f32),
            jax.ShapeDtypeStruct((_N, 16), _f32),
            jax.ShapeDtypeStruct((_N, 1), _f32),
        ],
    )(p_x, pos_n, pos_ca, pos_c, pos_cb, deg_parts, wg1)


def _block_body(last, acc_r, dinv_r, geom_r, bg_r, lng_r, lnb_r,
                wq_r, wk_r, wv_r, wo_r, bo_r, ag_r, ab_r, wg2_r, out_r):
    acc = jnp.concatenate([acc_r[q] for q in range(4)], axis=1)
    dinv = dinv_r[...]
    x = _ln(jnp.maximum(dinv * acc + bg_r[...], 0.0), lng_r[...], lnb_r[...])
    geom = geom_r[...]
    q = jnp.dot(x, wq_r[...], preferred_element_type=_f32)
    k = jnp.dot(x, wk_r[...], preferred_element_type=_f32)
    v = jnp.dot(x, wv_r[...], preferred_element_type=_f32)
    fns, fps, dss, fds = [], [], [], []
    for h in range(_H):
        qh = q[:, h * _QK:(h + 1) * _QK]
        kh = k[:, h * _QK:(h + 1) * _QK]
        vh = v[:, h * _VD:(h + 1) * _VD]
        logits = lax.dot_general(qh, kh, (((1,), (1,)), ((), ())),
                                 preferred_element_type=_f32)
        m = jnp.max(logits, axis=1, keepdims=True)
        e = jnp.exp(logits - m)
        alpha = e / jnp.sum(e, axis=1, keepdims=True)
        vcat = jnp.concatenate([vh, geom], axis=1)
        u = jnp.dot(alpha, vcat, preferred_element_type=_f32)
        fns.append(u[:, :_VD])
        apb = u[:, _VD + 0:_VD + 3] - geom[:, 3:6]
        dist = jnp.sqrt(jnp.sum(apb * apb, axis=1, keepdims=True))
        fp = jnp.concatenate(
            [jnp.sum(geom[:, 6 + 3 * i:9 + 3 * i] * apb, axis=1, keepdims=True)
             for i in range(3)], axis=1)
        fd = fp / (jnp.sqrt(jnp.sum(fp * fp, axis=1, keepdims=True)) + 1e-10)
        fns_last = None
        fps.append(fp)
        dss.append(dist)
        fds.append(fd)
    pad = jnp.zeros((_L, 128 - _OD), _f32)
    feat = jnp.concatenate(fns + fps + dss + fds + [pad], axis=1)
    fa = jnp.dot(feat, wo_r[...], preferred_element_type=_f32) + bo_r[...]
    px2 = x + _ln(x + fa, ag_r[...], ab_r[...])
    if last:
        out_r[...] = jnp.mean(px2, axis=0, keepdims=True)[None]
    else:
        y2 = jnp.dot(px2, wg2_r[...], preferred_element_type=_f32) * dinv
        for q in range(4):
            out_r[q] = y2[:, 32 * q:32 * q + 32]


def _block_call(last, acc, dinv, geom, bg, lng, lnb, wq, wk, wv, wo_pad, bo,
                ag, ab, wg2):
    if last:
        out_spec = pl.BlockSpec((1, 1, _DH), lambda b: (b, 0, 0))
        out_shape = jax.ShapeDtypeStruct((_B, 1, _DH), _f32)
    else:
        out_spec = pl.BlockSpec((4, _L, 32), lambda b: (0, b, 0))
        out_shape = jax.ShapeDtypeStruct((4, _N, 32), _f32)
    return pl.pallas_call(
        functools.partial(_block_body, last),
        grid=(_B,),
        in_specs=[
            pl.BlockSpec((4, _L, 32), lambda b: (0, b, 0)),
            pl.BlockSpec((_L, 1), lambda b: (b, 0)),
            pl.BlockSpec((_L, 16), lambda b: (b, 0)),
            pl.BlockSpec((1, _DH), lambda b: (0, 0)),
            pl.BlockSpec((1, _DH), lambda b: (0, 0)),
            pl.BlockSpec((1, _DH), lambda b: (0, 0)),
            pl.BlockSpec((_DH, _H * _QK), lambda b: (0, 0)),
            pl.BlockSpec((_DH, _H * _QK), lambda b: (0, 0)),
            pl.BlockSpec((_DH, _H * _VD), lambda b: (0, 0)),
            pl.BlockSpec((_DH, _DH), lambda b: (0, 0)),
            pl.BlockSpec((1, _DH), lambda b: (0, 0)),
            pl.BlockSpec((1, _DH), lambda b: (0, 0)),
            pl.BlockSpec((1, _DH), lambda b: (0, 0)),
            pl.BlockSpec((_DH, _DH), lambda b: (0, 0)),
        ],
        out_specs=out_spec,
        out_shape=out_shape,
    )(acc, dinv, geom, bg, lng, lnb, wq, wk, wv, wo_pad, bo, ag, ab, wg2)


def _head_body(gf_r, wf1_r, bf1_r, wf2_r, bf2_r, out_r):
    h = jnp.maximum(
        jnp.dot(gf_r[...], wf1_r[...], preferred_element_type=_f32)
        + bf1_r[...], 0.0)
    out_r[...] = (jnp.dot(h, wf2_r[...], preferred_element_type=_f32)
                  + bf2_r[...])


def _head_call(gf, wf1, bf1, wf2, bf2):
    return pl.pallas_call(
        _head_body,
        out_shape=jax.ShapeDtypeStruct((_B, _DH), _f32),
    )(gf, wf1, bf1, wf2, bf2)


def kernel(p_x, p_edge_index, pos_N, pos_CA, pos_C, pos_CB,
           Wg1, bg1, Wg2, bg2, ln1g, ln1b, ln2g, ln2b,
           Wq1, Wk1, Wv1, Wo1, bo1, ag1, ab1,
           Wq2, Wk2, Wv2, Wo2, bo2, ag2, ab2,
           Wf1, bf1, Wf2, bf2):
    src_r = p_edge_index[0].reshape(_ROWS, _CH)
    dst_r = p_edge_index[1].reshape(_ROWS, _CH)
    src_q = jnp.stack([src_r + q * _N for q in range(4)])

    deg_parts = _deg_call(dst_r).reshape(2, _N, 16)

    wo1p = jnp.concatenate([Wo1, jnp.zeros((128 - _OD, _DH), _f32)], axis=0)
    wo2p = jnp.concatenate([Wo2, jnp.zeros((128 - _OD, _DH), _f32)], axis=0)
    row = lambda t: t.reshape(1, -1)

    y1, geom, dinv = _prep_call(p_x, pos_N, pos_CA, pos_C, pos_CB,
                                deg_parts, Wg1)
    acc1 = _scat_call(y1.reshape(4 * _N, 32), src_q, dst_r)
    y2 = _block_call(False, acc1.reshape(4, _N, 32), dinv, geom,
                     row(bg1), row(ln1g), row(ln1b),
                     Wq1, Wk1, Wv1, wo1p, row(bo1), row(ag1), row(ab1), Wg2)
    acc2 = _scat_call(y2.reshape(4 * _N, 32), src_q, dst_r)
    gf = _block_call(True, acc2.reshape(4, _N, 32), dinv, geom,
                     row(bg2), row(ln2g), row(ln2b),
                     Wq2, Wk2, Wv2, wo2p, row(bo2), row(ag2), row(ab2), Wg2)
    return _head_call(gf.reshape(_B, _DH), Wf1, row(bf1), Wf2, row(bf2))


# speedup vs baseline: 13.3102x; 1.3110x over previous
import functools

import jax
import jax.numpy as jnp
from jax import lax
from jax.experimental import pallas as pl
from jax.experimental.pallas import tpu as pltpu
from jax.experimental.pallas import tpu_sc as plsc

_B, _L, _H, _QK, _VD = 64, 256, 4, 16, 16
_N = _B * _L
_E = 524288
_DIN, _DH = 33, 128
_OD = _H * _VD + _H * 7

_NC, _NS = 2, 16
_CH = 128
_ROWS = _E // _CH
_NLOC = _N // _NS

_f32 = jnp.float32


@functools.lru_cache(maxsize=None)
def _sc_mesh():
    return plsc.VectorSubcoreMesh(core_axis_name="c", subcore_axis_name="s",
                                  num_cores=_NC, num_subcores=_NS)


def _deg_body(dst_r, out, idx_v, zbuf, obuf, acc):
    c = lax.axis_index("c")
    s = lax.axis_index("s")
    w = c * _NS + s
    rpw = _ROWS // (_NC * _NS)
    pltpu.sync_copy(dst_r.at[pl.ds(w * rpw, rpw)], idx_v)

    def fill(i, _):
        zbuf[i] = jnp.zeros((16,), _f32)
        obuf[i] = jnp.full((16,), 1.0, _f32)
        return 0

    lax.fori_loop(0, _CH, fill, 0)
    for kk in range(_NLOC // _CH):
        pltpu.sync_copy(zbuf, acc.at[pl.ds(s * _NLOC + kk * _CH, _CH)])
    plsc.subcore_barrier()

    def step(j, _):
        pltpu.sync_copy(obuf, acc.at[idx_v.at[j]], add=True)
        return 0

    lax.fori_loop(0, rpw, step, 0)
    plsc.subcore_barrier()
    pltpu.sync_copy(acc.at[pl.ds(s * _NLOC, _NLOC)],
                    out.at[pl.ds(c * _N + s * _NLOC, _NLOC)])


@functools.lru_cache(maxsize=None)
def _deg_kernel():
    return pl.kernel(
        _deg_body,
        out_type=jax.ShapeDtypeStruct((_NC * _N, 16), _f32),
        mesh=_sc_mesh(),
        scratch_types=[
            pltpu.VMEM((_ROWS // (_NC * _NS), _CH), jnp.int32),
            pltpu.VMEM((_CH, 16), _f32),
            pltpu.VMEM((_CH, 16), _f32),
            pltpu.VMEM_SHARED((_N, 16), _f32),
        ],
        compiler_params=pltpu.CompilerParams(use_tc_tiling_on_sc=False),
    )


def _deg_call(dst_r):
    return _deg_kernel()(dst_r)


def _scat_body(y4, src_q, dst_r, out, isrc, idst, buf0, buf1, acc,
               sem0, sem1):
    c = lax.axis_index("c")
    s = lax.axis_index("s")
    rps = _ROWS // _NS
    pltpu.sync_copy(dst_r.at[pl.ds(s * rps, rps)], idst)
    for j in range(2):
        q = 2 * c + j
        pltpu.sync_copy(src_q.at[q, pl.ds(s * rps, rps)], isrc)
        pltpu.sync_copy(y4.at[pl.ds(q * _N + s * _NLOC, _NLOC)],
                        acc.at[pl.ds(s * _NLOC, _NLOC)])
        plsc.subcore_barrier()

        pltpu.async_copy(y4.at[isrc.at[0]], buf0, sem0)

        def step(t, _):
            i = 2 * t
            pltpu.async_copy(y4.at[isrc.at[i + 1]], buf1, sem1)
            pltpu.make_async_copy(y4.at[isrc.at[i]], buf0, sem0).wait()
            pltpu.sync_copy(buf0, acc.at[idst.at[i]], add=True)
            i2 = jnp.minimum(i + 2, rps - 1)
            pltpu.async_copy(y4.at[isrc.at[i2]], buf0, sem0)
            pltpu.make_async_copy(y4.at[isrc.at[i + 1]], buf1, sem1).wait()
            pltpu.sync_copy(buf1, acc.at[idst.at[i + 1]], add=True)
            return 0

        lax.fori_loop(0, rps // 2, step, 0)
        pltpu.make_async_copy(y4.at[isrc.at[rps - 1]], buf0, sem0).wait()
        plsc.subcore_barrier()
        pltpu.sync_copy(acc.at[pl.ds(s * _NLOC, _NLOC)],
                        out.at[pl.ds(q * _N + s * _NLOC, _NLOC)])


@functools.lru_cache(maxsize=None)
def _scat_kernel():
    return pl.kernel(
        _scat_body,
        out_type=jax.ShapeDtypeStruct((4 * _N, 32), _f32),
        mesh=_sc_mesh(),
        scratch_types=[
            pltpu.VMEM((_ROWS // _NS, _CH), jnp.int32),
            pltpu.VMEM((_ROWS // _NS, _CH), jnp.int32),
            pltpu.VMEM((_CH, 32), _f32),
            pltpu.VMEM((_CH, 32), _f32),
            pltpu.VMEM_SHARED((_N, 32), _f32),
            pltpu.SemaphoreType.DMA,
            pltpu.SemaphoreType.DMA,
        ],
        compiler_params=pltpu.CompilerParams(use_tc_tiling_on_sc=False),
    )


def _scat_call(y4, src_q, dst_r):
    return _scat_kernel()(y4, src_q, dst_r)


def _ln(x, g, b):
    m = jnp.mean(x, axis=-1, keepdims=True)
    v = jnp.mean((x - m) * (x - m), axis=-1, keepdims=True)
    return (x - m) / jnp.sqrt(v + 1e-5) * g + b


def _prep_body(px_r, pn_r, pca_r, pc_r, pcb_r, dp_r, wg1_r,
               y_r, geom_r, dinv_r):
    deg = dp_r[0, :, 0:1] + dp_r[1, :, 0:1] + 1.0
    dinv = lax.rsqrt(deg)
    dinv_r[...] = dinv
    h = jnp.dot(px_r[...], wg1_r[...], preferred_element_type=_f32)
    y = h * dinv
    for q in range(4):
        y_r[q] = y[:, 32 * q:32 * q + 32]
    pn, pca, pc, pcb = pn_r[...], pca_r[...], pc_r[...], pcb_r[...]
    e0 = pn - pca
    e1 = pc - pca
    e0 = e0 / jnp.sqrt(jnp.sum(e0 * e0, axis=1, keepdims=True) + 1e-10)
    dot = jnp.sum(e0 * e1, axis=1, keepdims=True)
    e1 = e1 - e0 * dot
    e1 = e1 / jnp.sqrt(jnp.sum(e1 * e1, axis=1, keepdims=True) + 1e-10)
    a0, a1, a2 = e0[:, 0:1], e0[:, 1:2], e0[:, 2:3]
    b0, b1, b2 = e1[:, 0:1], e1[:, 1:2], e1[:, 2:3]
    e2 = jnp.concatenate(
        [a1 * b2 - a2 * b1, a2 * b0 - a0 * b2, a0 * b1 - a1 * b0], axis=1)
    zero = jnp.zeros((pn.shape[0], 1), _f32)
    geom_r[...] = jnp.concatenate([pcb, pca, e0, e1, e2, zero], axis=1)


def _prep_call(p_x, pos_n, pos_ca, pos_c, pos_cb, deg_parts, wg1):
    nb = 16
    blk = _N // nb
    return pl.pallas_call(
        _prep_body,
        grid=(nb,),
        in_specs=[
            pl.BlockSpec((blk, _DIN), lambda i: (i, 0)),
            pl.BlockSpec((blk, 3), lambda i: (i, 0)),
            pl.BlockSpec((blk, 3), lambda i: (i, 0)),
            pl.BlockSpec((blk, 3), lambda i: (i, 0)),
            pl.BlockSpec((blk, 3), lambda i: (i, 0)),
            pl.BlockSpec((2, blk, 16), lambda i: (0, i, 0)),
            pl.BlockSpec((_DIN, _DH), lambda i: (0, 0)),
        ],
        out_specs=[
            pl.BlockSpec((4, blk, 32), lambda i: (0, i, 0)),
            pl.BlockSpec((blk, 16), lambda i: (i, 0)),
            pl.BlockSpec((blk, 1), lambda i: (i, 0)),
        ],
        out_shape=[
            jax.ShapeDtypeStruct((4, _N, 32), _f32),
            jax.ShapeDtypeStruct((_N, 16), _f32),
            jax.ShapeDtypeStruct((_N, 1), _f32),
        ],
    )(p_x, pos_n, pos_ca, pos_c, pos_cb, deg_parts, wg1)


def _block_body(last, acc_r, dinv_r, geom_r, bg_r, lng_r, lnb_r,
                wq_r, wk_r, wv_r, wo_r, bo_r, ag_r, ab_r, wg2_r, out_r):
    acc = jnp.concatenate([acc_r[q] for q in range(4)], axis=1)
    dinv = dinv_r[...]
    x = _ln(jnp.maximum(dinv * acc + bg_r[...], 0.0), lng_r[...], lnb_r[...])
    geom = geom_r[...]
    q = jnp.dot(x, wq_r[...], preferred_element_type=_f32)
    k = jnp.dot(x, wk_r[...], preferred_element_type=_f32)
    v = jnp.dot(x, wv_r[...], preferred_element_type=_f32)
    fns, fps, dss, fds = [], [], [], []
    for h in range(_H):
        qh = q[:, h * _QK:(h + 1) * _QK]
        kh = k[:, h * _QK:(h + 1) * _QK]
        vh = v[:, h * _VD:(h + 1) * _VD]
        logits = lax.dot_general(qh, kh, (((1,), (1,)), ((), ())),
                                 preferred_element_type=_f32)
        m = jnp.max(logits, axis=1, keepdims=True)
        e = jnp.exp(logits - m)
        alpha = e / jnp.sum(e, axis=1, keepdims=True)
        vcat = jnp.concatenate([vh, geom], axis=1)
        u = jnp.dot(alpha, vcat, preferred_element_type=_f32)
        fns.append(u[:, :_VD])
        apb = u[:, _VD + 0:_VD + 3] - geom[:, 3:6]
        dist = jnp.sqrt(jnp.sum(apb * apb, axis=1, keepdims=True))
        fp = jnp.concatenate(
            [jnp.sum(geom[:, 6 + 3 * i:9 + 3 * i] * apb, axis=1, keepdims=True)
             for i in range(3)], axis=1)
        fd = fp / (jnp.sqrt(jnp.sum(fp * fp, axis=1, keepdims=True)) + 1e-10)
        fns_last = None
        fps.append(fp)
        dss.append(dist)
        fds.append(fd)
    pad = jnp.zeros((_L, 128 - _OD), _f32)
    feat = jnp.concatenate(fns + fps + dss + fds + [pad], axis=1)
    fa = jnp.dot(feat, wo_r[...], preferred_element_type=_f32) + bo_r[...]
    px2 = x + _ln(x + fa, ag_r[...], ab_r[...])
    if last:
        out_r[...] = jnp.mean(px2, axis=0, keepdims=True)[None]
    else:
        y2 = jnp.dot(px2, wg2_r[...], preferred_element_type=_f32) * dinv
        for q in range(4):
            out_r[q] = y2[:, 32 * q:32 * q + 32]


def _block_call(last, acc, dinv, geom, bg, lng, lnb, wq, wk, wv, wo_pad, bo,
                ag, ab, wg2):
    if last:
        out_spec = pl.BlockSpec((1, 1, _DH), lambda b: (b, 0, 0))
        out_shape = jax.ShapeDtypeStruct((_B, 1, _DH), _f32)
    else:
        out_spec = pl.BlockSpec((4, _L, 32), lambda b: (0, b, 0))
        out_shape = jax.ShapeDtypeStruct((4, _N, 32), _f32)
    return pl.pallas_call(
        functools.partial(_block_body, last),
        grid=(_B,),
        in_specs=[
            pl.BlockSpec((4, _L, 32), lambda b: (0, b, 0)),
            pl.BlockSpec((_L, 1), lambda b: (b, 0)),
            pl.BlockSpec((_L, 16), lambda b: (b, 0)),
            pl.BlockSpec((1, _DH), lambda b: (0, 0)),
            pl.BlockSpec((1, _DH), lambda b: (0, 0)),
            pl.BlockSpec((1, _DH), lambda b: (0, 0)),
            pl.BlockSpec((_DH, _H * _QK), lambda b: (0, 0)),
            pl.BlockSpec((_DH, _H * _QK), lambda b: (0, 0)),
            pl.BlockSpec((_DH, _H * _VD), lambda b: (0, 0)),
            pl.BlockSpec((_DH, _DH), lambda b: (0, 0)),
            pl.BlockSpec((1, _DH), lambda b: (0, 0)),
            pl.BlockSpec((1, _DH), lambda b: (0, 0)),
            pl.BlockSpec((1, _DH), lambda b: (0, 0)),
            pl.BlockSpec((_DH, _DH), lambda b: (0, 0)),
        ],
        out_specs=out_spec,
        out_shape=out_shape,
    )(acc, dinv, geom, bg, lng, lnb, wq, wk, wv, wo_pad, bo, ag, ab, wg2)


def _head_body(gf_r, wf1_r, bf1_r, wf2_r, bf2_r, out_r):
    h = jnp.maximum(
        jnp.dot(gf_r[...], wf1_r[...], preferred_element_type=_f32)
        + bf1_r[...], 0.0)
    out_r[...] = (jnp.dot(h, wf2_r[...], preferred_element_type=_f32)
                  + bf2_r[...])


def _head_call(gf, wf1, bf1, wf2, bf2):
    return pl.pallas_call(
        _head_body,
        out_shape=jax.ShapeDtypeStruct((_B, _DH), _f32),
    )(gf, wf1, bf1, wf2, bf2)


def kernel(p_x, p_edge_index, pos_N, pos_CA, pos_C, pos_CB,
           Wg1, bg1, Wg2, bg2, ln1g, ln1b, ln2g, ln2b,
           Wq1, Wk1, Wv1, Wo1, bo1, ag1, ab1,
           Wq2, Wk2, Wv2, Wo2, bo2, ag2, ab2,
           Wf1, bf1, Wf2, bf2):
    src_r = p_edge_index[0].reshape(_ROWS, _CH)
    dst_r = p_edge_index[1].reshape(_ROWS, _CH)
    src_q = jnp.stack([src_r + q * _N for q in range(4)])

    deg_parts = _deg_call(dst_r).reshape(2, _N, 16)

    wo1p = jnp.concatenate([Wo1, jnp.zeros((128 - _OD, _DH), _f32)], axis=0)
    wo2p = jnp.concatenate([Wo2, jnp.zeros((128 - _OD, _DH), _f32)], axis=0)
    row = lambda t: t.reshape(1, -1)

    y1, geom, dinv = _prep_call(p_x, pos_N, pos_CA, pos_C, pos_CB,
                                deg_parts, Wg1)
    acc1 = _scat_call(y1.reshape(4 * _N, 32), src_q, dst_r)
    y2 = _block_call(False, acc1.reshape(4, _N, 32), dinv, geom,
                     row(bg1), row(ln1g), row(ln1b),
                     Wq1, Wk1, Wv1, wo1p, row(bo1), row(ag1), row(ab1), Wg2)
    acc2 = _scat_call(y2.reshape(4 * _N, 32), src_q, dst_r)
    gf = _block_call(True, acc2.reshape(4, _N, 32), dinv, geom,
                     row(bg2), row(ln2g), row(ln2b),
                     Wq2, Wk2, Wv2, wo2p, row(bo2), row(ag2), row(ab2), Wg2)
    return _head_call(gf.reshape(_B, _DH), Wf1, row(bf1), Wf2, row(bf2))


# speedup vs baseline: 18.6186x; 1.3988x over previous
import functools

import jax
import jax.numpy as jnp
from jax import lax
from jax.experimental import pallas as pl
from jax.experimental.pallas import tpu as pltpu
from jax.experimental.pallas import tpu_sc as plsc

_B, _L, _H, _QK, _VD = 64, 256, 4, 16, 16
_N = _B * _L
_E = 524288
_DIN, _DH = 33, 128
_OD = _H * _VD + _H * 7

_NC, _NS = 2, 16
_CH = 128
_ROWS = _E // _CH
_NLOC = _N // _NS

_f32 = jnp.float32


@functools.lru_cache(maxsize=None)
def _sc_mesh():
    return plsc.VectorSubcoreMesh(core_axis_name="c", subcore_axis_name="s",
                                  num_cores=_NC, num_subcores=_NS)


def _deg_body(dst_r, out, idx_v, zbuf, obuf, acc):
    c = lax.axis_index("c")
    s = lax.axis_index("s")
    w = c * _NS + s
    rpw = _ROWS // (_NC * _NS)
    pltpu.sync_copy(dst_r.at[pl.ds(w * rpw, rpw)], idx_v)

    def fill(i, _):
        zbuf[i] = jnp.zeros((16,), _f32)
        obuf[i] = jnp.full((16,), 1.0, _f32)
        return 0

    lax.fori_loop(0, _CH, fill, 0)
    for kk in range(_NLOC // _CH):
        pltpu.sync_copy(zbuf, acc.at[pl.ds(s * _NLOC + kk * _CH, _CH)])
    plsc.subcore_barrier()

    def step(j, _):
        pltpu.sync_copy(obuf, acc.at[idx_v.at[j]], add=True)
        return 0

    lax.fori_loop(0, rpw, step, 0)
    plsc.subcore_barrier()
    pltpu.sync_copy(acc.at[pl.ds(s * _NLOC, _NLOC)],
                    out.at[pl.ds(c * _N + s * _NLOC, _NLOC)])


@functools.lru_cache(maxsize=None)
def _deg_kernel():
    return pl.kernel(
        _deg_body,
        out_type=jax.ShapeDtypeStruct((_NC * _N, 16), _f32),
        mesh=_sc_mesh(),
        scratch_types=[
            pltpu.VMEM((_ROWS // (_NC * _NS), _CH), jnp.int32),
            pltpu.VMEM((_CH, 16), _f32),
            pltpu.VMEM((_CH, 16), _f32),
            pltpu.VMEM_SHARED((_N, 16), _f32),
        ],
        compiler_params=pltpu.CompilerParams(use_tc_tiling_on_sc=False),
    )


def _deg_call(dst_r):
    return _deg_kernel()(dst_r)


def _scat_body(y4, src_q, dst_r, out, isrc, idst, buf0, buf1, acc,
               sem0, sem1):
    c = lax.axis_index("c")
    s = lax.axis_index("s")
    rps = _ROWS // _NS
    pltpu.sync_copy(dst_r.at[pl.ds(s * rps, rps)], idst)
    for j in range(2):
        q = 2 * c + j
        pltpu.sync_copy(src_q.at[q, pl.ds(s * rps, rps)], isrc)
        pltpu.sync_copy(y4.at[pl.ds(q * _N + s * _NLOC, _NLOC)],
                        acc.at[pl.ds(s * _NLOC, _NLOC)])
        plsc.subcore_barrier()

        pltpu.async_copy(y4.at[isrc.at[0]], buf0, sem0)

        def step(t, _):
            i = 2 * t
            pltpu.async_copy(y4.at[isrc.at[i + 1]], buf1, sem1)
            pltpu.make_async_copy(y4.at[isrc.at[i]], buf0, sem0).wait()
            pltpu.sync_copy(buf0, acc.at[idst.at[i]], add=True)
            i2 = jnp.minimum(i + 2, rps - 1)
            pltpu.async_copy(y4.at[isrc.at[i2]], buf0, sem0)
            pltpu.make_async_copy(y4.at[isrc.at[i + 1]], buf1, sem1).wait()
            pltpu.sync_copy(buf1, acc.at[idst.at[i + 1]], add=True)
            return 0

        lax.fori_loop(0, rps // 2, step, 0)
        pltpu.make_async_copy(y4.at[isrc.at[rps - 1]], buf0, sem0).wait()
        plsc.subcore_barrier()
        pltpu.sync_copy(acc.at[pl.ds(s * _NLOC, _NLOC)],
                        out.at[pl.ds(q * _N + s * _NLOC, _NLOC)])


@functools.lru_cache(maxsize=None)
def _scat_kernel():
    return pl.kernel(
        _scat_body,
        out_type=jax.ShapeDtypeStruct((4 * _N, 32), _f32),
        mesh=_sc_mesh(),
        scratch_types=[
            pltpu.VMEM((_ROWS // _NS, _CH), jnp.int32),
            pltpu.VMEM((_ROWS // _NS, _CH), jnp.int32),
            pltpu.VMEM((_CH, 32), _f32),
            pltpu.VMEM((_CH, 32), _f32),
            pltpu.VMEM_SHARED((_N, 32), _f32),
            pltpu.SemaphoreType.DMA,
            pltpu.SemaphoreType.DMA,
        ],
        compiler_params=pltpu.CompilerParams(use_tc_tiling_on_sc=False),
    )


def _scat_call(y4, src_q, dst_r):
    return _scat_kernel()(y4, src_q, dst_r)


def _ln(x, g, b):
    m = jnp.mean(x, axis=-1, keepdims=True)
    v = jnp.mean((x - m) * (x - m), axis=-1, keepdims=True)
    return (x - m) / jnp.sqrt(v + 1e-5) * g + b


def _prep_body(px_r, pn_r, pca_r, pc_r, pcb_r, dp_r, wg1_r,
               y_r, geom_r, dinv_r):
    deg = dp_r[0, :, 0:1] + dp_r[1, :, 0:1] + 1.0
    dinv = lax.rsqrt(deg)
    dinv_r[...] = dinv
    h = jnp.dot(px_r[...], wg1_r[...], preferred_element_type=_f32)
    y = h * dinv
    for q in range(4):
        y_r[q] = y[:, 32 * q:32 * q + 32]
    pn, pca, pc, pcb = pn_r[...], pca_r[...], pc_r[...], pcb_r[...]
    e0 = pn - pca
    e1 = pc - pca
    e0 = e0 / jnp.sqrt(jnp.sum(e0 * e0, axis=1, keepdims=True) + 1e-10)
    dot = jnp.sum(e0 * e1, axis=1, keepdims=True)
    e1 = e1 - e0 * dot
    e1 = e1 / jnp.sqrt(jnp.sum(e1 * e1, axis=1, keepdims=True) + 1e-10)
    a0, a1, a2 = e0[:, 0:1], e0[:, 1:2], e0[:, 2:3]
    b0, b1, b2 = e1[:, 0:1], e1[:, 1:2], e1[:, 2:3]
    e2 = jnp.concatenate(
        [a1 * b2 - a2 * b1, a2 * b0 - a0 * b2, a0 * b1 - a1 * b0], axis=1)
    zero = jnp.zeros((pn.shape[0], 1), _f32)
    geom_r[...] = jnp.concatenate([pcb, pca, e0, e1, e2, zero], axis=1)


def _prep_call(p_x, pos_n, pos_ca, pos_c, pos_cb, deg_parts, wg1):
    nb = 16
    blk = _N // nb
    return pl.pallas_call(
        _prep_body,
        grid=(nb,),
        in_specs=[
            pl.BlockSpec((blk, _DIN), lambda i: (i, 0)),
            pl.BlockSpec((blk, 3), lambda i: (i, 0)),
            pl.BlockSpec((blk, 3), lambda i: (i, 0)),
            pl.BlockSpec((blk, 3), lambda i: (i, 0)),
            pl.BlockSpec((blk, 3), lambda i: (i, 0)),
            pl.BlockSpec((2, blk, 16), lambda i: (0, i, 0)),
            pl.BlockSpec((_DIN, _DH), lambda i: (0, 0)),
        ],
        out_specs=[
            pl.BlockSpec((4, blk, 32), lambda i: (0, i, 0)),
            pl.BlockSpec((blk, 16), lambda i: (i, 0)),
            pl.BlockSpec((blk, 1), lambda i: (i, 0)),
        ],
        out_shape=[
            jax.ShapeDtypeStruct((4, _N, 32), _f32),
            jax.ShapeDtypeStruct((_N, 16), _f32),
            jax.ShapeDtypeStruct((_N, 1), _f32),
        ],
    )(p_x, pos_n, pos_ca, pos_c, pos_cb, deg_parts, wg1)


def _block_body(last, acc_r, dinv_r, geom_r, bg_r, lng_r, lnb_r,
                wq_r, wk_r, wv_r, wo_r, bo_r, ag_r, ab_r, wg2_r, out_r):
    acc = jnp.concatenate([acc_r[q] for q in range(4)], axis=1)
    dinv = dinv_r[...]
    x = _ln(jnp.maximum(dinv * acc + bg_r[...], 0.0), lng_r[...], lnb_r[...])
    geom = geom_r[...]
    geom_t = jnp.transpose(geom)
    q = jnp.dot(x, wq_r[...], preferred_element_type=_f32)
    k = jnp.dot(x, wk_r[...], preferred_element_type=_f32)
    v = jnp.dot(x, wv_r[...], preferred_element_type=_f32)
    fns, fps, dss, fds = [], [], [], []
    for h in range(_H):
        qh = q[:, h * _QK:(h + 1) * _QK]
        kh = k[:, h * _QK:(h + 1) * _QK]
        vh = v[:, h * _VD:(h + 1) * _VD]
        logits_t = lax.dot_general(kh, qh, (((1,), (1,)), ((), ())),
                                   preferred_element_type=_f32)
        m = jnp.max(logits_t, axis=0, keepdims=True)
        e = jnp.exp(logits_t - m)
        alpha_t = e / jnp.sum(e, axis=0, keepdims=True)
        vcat = jnp.concatenate([vh, geom], axis=1)
        u_t = lax.dot_general(vcat, alpha_t, (((0,), (0,)), ((), ())),
                              preferred_element_type=_f32)
        fns.append(u_t[:_VD])
        apb_t = u_t[_VD:_VD + 3] - geom_t[3:6]
        dist_t = jnp.sqrt(jnp.sum(apb_t * apb_t, axis=0, keepdims=True))
        fp_t = jnp.concatenate(
            [jnp.sum(geom_t[6 + 3 * i:9 + 3 * i] * apb_t, axis=0,
                     keepdims=True) for i in range(3)], axis=0)
        fd_t = fp_t / (jnp.sqrt(jnp.sum(fp_t * fp_t, axis=0, keepdims=True))
                       + 1e-10)
        fps.append(fp_t)
        dss.append(dist_t)
        fds.append(fd_t)
    pad = jnp.zeros((128 - _OD, _L), _f32)
    feat_t = jnp.concatenate(fns + fps + dss + fds + [pad], axis=0)
    fa = lax.dot_general(feat_t, wo_r[...], (((0,), (0,)), ((), ())),
                         preferred_element_type=_f32) + bo_r[...]
    px2 = x + _ln(x + fa, ag_r[...], ab_r[...])
    if last:
        out_r[...] = jnp.mean(px2, axis=0, keepdims=True)[None]
    else:
        y2 = jnp.dot(px2, wg2_r[...], preferred_element_type=_f32) * dinv
        for q in range(4):
            out_r[q] = y2[:, 32 * q:32 * q + 32]


def _block_call(last, acc, dinv, geom, bg, lng, lnb, wq, wk, wv, wo_pad, bo,
                ag, ab, wg2):
    if last:
        out_spec = pl.BlockSpec((1, 1, _DH), lambda b: (b, 0, 0))
        out_shape = jax.ShapeDtypeStruct((_B, 1, _DH), _f32)
    else:
        out_spec = pl.BlockSpec((4, _L, 32), lambda b: (0, b, 0))
        out_shape = jax.ShapeDtypeStruct((4, _N, 32), _f32)
    return pl.pallas_call(
        functools.partial(_block_body, last),
        grid=(_B,),
        in_specs=[
            pl.BlockSpec((4, _L, 32), lambda b: (0, b, 0)),
            pl.BlockSpec((_L, 1), lambda b: (b, 0)),
            pl.BlockSpec((_L, 16), lambda b: (b, 0)),
            pl.BlockSpec((1, _DH), lambda b: (0, 0)),
            pl.BlockSpec((1, _DH), lambda b: (0, 0)),
            pl.BlockSpec((1, _DH), lambda b: (0, 0)),
            pl.BlockSpec((_DH, _H * _QK), lambda b: (0, 0)),
            pl.BlockSpec((_DH, _H * _QK), lambda b: (0, 0)),
            pl.BlockSpec((_DH, _H * _VD), lambda b: (0, 0)),
            pl.BlockSpec((_DH, _DH), lambda b: (0, 0)),
            pl.BlockSpec((1, _DH), lambda b: (0, 0)),
            pl.BlockSpec((1, _DH), lambda b: (0, 0)),
            pl.BlockSpec((1, _DH), lambda b: (0, 0)),
            pl.BlockSpec((_DH, _DH), lambda b: (0, 0)),
        ],
        out_specs=out_spec,
        out_shape=out_shape,
    )(acc, dinv, geom, bg, lng, lnb, wq, wk, wv, wo_pad, bo, ag, ab, wg2)


def _head_body(gf_r, wf1_r, bf1_r, wf2_r, bf2_r, out_r):
    h = jnp.maximum(
        jnp.dot(gf_r[...], wf1_r[...], preferred_element_type=_f32)
        + bf1_r[...], 0.0)
    out_r[...] = (jnp.dot(h, wf2_r[...], preferred_element_type=_f32)
                  + bf2_r[...])


def _head_call(gf, wf1, bf1, wf2, bf2):
    return pl.pallas_call(
        _head_body,
        out_shape=jax.ShapeDtypeStruct((_B, _DH), _f32),
    )(gf, wf1, bf1, wf2, bf2)


def kernel(p_x, p_edge_index, pos_N, pos_CA, pos_C, pos_CB,
           Wg1, bg1, Wg2, bg2, ln1g, ln1b, ln2g, ln2b,
           Wq1, Wk1, Wv1, Wo1, bo1, ag1, ab1,
           Wq2, Wk2, Wv2, Wo2, bo2, ag2, ab2,
           Wf1, bf1, Wf2, bf2):
    src_r = p_edge_index[0].reshape(_ROWS, _CH)
    dst_r = p_edge_index[1].reshape(_ROWS, _CH)
    src_q = jnp.stack([src_r + q * _N for q in range(4)])

    deg_parts = _deg_call(dst_r).reshape(2, _N, 16)

    wo1p = jnp.concatenate([Wo1, jnp.zeros((128 - _OD, _DH), _f32)], axis=0)
    wo2p = jnp.concatenate([Wo2, jnp.zeros((128 - _OD, _DH), _f32)], axis=0)
    row = lambda t: t.reshape(1, -1)

    y1, geom, dinv = _prep_call(p_x, pos_N, pos_CA, pos_C, pos_CB,
                                deg_parts, Wg1)
    acc1 = _scat_call(y1.reshape(4 * _N, 32), src_q, dst_r)
    y2 = _block_call(False, acc1.reshape(4, _N, 32), dinv, geom,
                     row(bg1), row(ln1g), row(ln1b),
                     Wq1, Wk1, Wv1, wo1p, row(bo1), row(ag1), row(ab1), Wg2)
    acc2 = _scat_call(y2.reshape(4 * _N, 32), src_q, dst_r)
    gf = _block_call(True, acc2.reshape(4, _N, 32), dinv, geom,
                     row(bg2), row(ln2g), row(ln2b),
                     Wq2, Wk2, Wv2, wo2p, row(bo2), row(ag2), row(ab2), Wg2)
    return _head_call(gf.reshape(_B, _DH), Wf1, row(bf1), Wf2, row(bf2))


# speedup vs baseline: 20.0233x; 1.0754x over previous
import functools

import jax
import jax.numpy as jnp
from jax import lax
from jax.experimental import pallas as pl
from jax.experimental.pallas import tpu as pltpu
from jax.experimental.pallas import tpu_sc as plsc

_B, _L, _H, _QK, _VD = 64, 256, 4, 16, 16
_N = _B * _L
_E = 524288
_DIN, _DH = 33, 128
_OD = _H * _VD + _H * 7

_NC, _NS = 2, 16
_CH = 128
_ROWS = _E // _CH
_NLOC = _N // _NS

_f32 = jnp.float32


@functools.lru_cache(maxsize=None)
def _sc_mesh():
    return plsc.VectorSubcoreMesh(core_axis_name="c", subcore_axis_name="s",
                                  num_cores=_NC, num_subcores=_NS)


def _deg_body(dst_r, out, idx_v, zbuf, obuf, acc):
    c = lax.axis_index("c")
    s = lax.axis_index("s")
    w = c * _NS + s
    rpw = _ROWS // (_NC * _NS)
    pltpu.sync_copy(dst_r.at[pl.ds(w * rpw, rpw)], idx_v)

    def fill(i, _):
        zbuf[i] = jnp.zeros((16,), _f32)
        obuf[i] = jnp.full((16,), 1.0, _f32)
        return 0

    lax.fori_loop(0, _CH, fill, 0)
    for kk in range(_NLOC // _CH):
        pltpu.sync_copy(zbuf, acc.at[pl.ds(s * _NLOC + kk * _CH, _CH)])
    plsc.subcore_barrier()

    def step(j, _):
        pltpu.sync_copy(obuf, acc.at[idx_v.at[j]], add=True)
        return 0

    lax.fori_loop(0, rpw, step, 0)
    plsc.subcore_barrier()
    pltpu.sync_copy(acc.at[pl.ds(s * _NLOC, _NLOC)],
                    out.at[pl.ds(c * _N + s * _NLOC, _NLOC)])


@functools.lru_cache(maxsize=None)
def _deg_kernel():
    return pl.kernel(
        _deg_body,
        out_type=jax.ShapeDtypeStruct((_NC * _N, 16), _f32),
        mesh=_sc_mesh(),
        scratch_types=[
            pltpu.VMEM((_ROWS // (_NC * _NS), _CH), jnp.int32),
            pltpu.VMEM((_CH, 16), _f32),
            pltpu.VMEM((_CH, 16), _f32),
            pltpu.VMEM_SHARED((_N, 16), _f32),
        ],
        compiler_params=pltpu.CompilerParams(use_tc_tiling_on_sc=False),
    )


def _deg_call(dst_r):
    return _deg_kernel()(dst_r)


def _scat_body(y4, src_q, dst_r, out, isrc, idst,
               b0, b1, b2_, b3, acc,
               g0, g1, g2, g3, s0, s1, s2, s3):
    bufs = (b0, b1, b2_, b3)
    gsem = (g0, g1, g2, g3)
    ssem = (s0, s1, s2, s3)
    c = lax.axis_index("c")
    s = lax.axis_index("s")
    rps = _ROWS // _NS
    pltpu.sync_copy(dst_r.at[pl.ds(s * rps, rps)], idst)
    for j in range(2):
        q = 2 * c + j
        pltpu.sync_copy(src_q.at[q, pl.ds(s * rps, rps)], isrc)
        pltpu.sync_copy(y4.at[pl.ds(q * _N + s * _NLOC, _NLOC)],
                        acc.at[pl.ds(s * _NLOC, _NLOC)])
        plsc.subcore_barrier()

        pltpu.async_copy(y4.at[isrc.at[0]], bufs[0], gsem[0])
        pltpu.async_copy(y4.at[isrc.at[1]], bufs[1], gsem[1])

        def step(t, _):
            for b in range(4):
                i = 4 * t + b
                b2 = (b + 2) % 4
                pltpu.make_async_copy(y4.at[isrc.at[i]], bufs[b],
                                      gsem[b]).wait()
                pltpu.async_copy(bufs[b], acc.at[idst.at[i]], ssem[b],
                                 add=True)
                if b < 2:
                    @pl.when(t > 0)
                    def _():
                        pltpu.make_async_copy(
                            bufs[b2], acc.at[idst.at[0]], ssem[b2]).wait()
                else:
                    pltpu.make_async_copy(
                        bufs[b2], acc.at[idst.at[0]], ssem[b2]).wait()
                i2 = jnp.minimum(i + 2, rps - 1)
                pltpu.async_copy(y4.at[isrc.at[i2]], bufs[b2], gsem[b2])
            return 0

        lax.fori_loop(0, rps // 4, step, 0)
        pltpu.make_async_copy(y4.at[isrc.at[0]], bufs[0], gsem[0]).wait()
        pltpu.make_async_copy(y4.at[isrc.at[0]], bufs[1], gsem[1]).wait()
        pltpu.make_async_copy(bufs[2], acc.at[idst.at[0]], ssem[2]).wait()
        pltpu.make_async_copy(bufs[3], acc.at[idst.at[0]], ssem[3]).wait()
        plsc.subcore_barrier()
        pltpu.sync_copy(acc.at[pl.ds(s * _NLOC, _NLOC)],
                        out.at[pl.ds(q * _N + s * _NLOC, _NLOC)])


@functools.lru_cache(maxsize=None)
def _scat_kernel():
    return pl.kernel(
        _scat_body,
        out_type=jax.ShapeDtypeStruct((4 * _N, 32), _f32),
        mesh=_sc_mesh(),
        scratch_types=[
            pltpu.VMEM((_ROWS // _NS, _CH), jnp.int32),
            pltpu.VMEM((_ROWS // _NS, _CH), jnp.int32),
            pltpu.VMEM((_CH, 32), _f32),
            pltpu.VMEM((_CH, 32), _f32),
            pltpu.VMEM((_CH, 32), _f32),
            pltpu.VMEM((_CH, 32), _f32),
            pltpu.VMEM_SHARED((_N, 32), _f32),
            pltpu.SemaphoreType.DMA,
            pltpu.SemaphoreType.DMA,
            pltpu.SemaphoreType.DMA,
            pltpu.SemaphoreType.DMA,
            pltpu.SemaphoreType.DMA,
            pltpu.SemaphoreType.DMA,
            pltpu.SemaphoreType.DMA,
            pltpu.SemaphoreType.DMA,
        ],
        compiler_params=pltpu.CompilerParams(use_tc_tiling_on_sc=False),
    )


def _scat_call(y4, src_q, dst_r):
    return _scat_kernel()(y4, src_q, dst_r)


def _ln(x, g, b):
    m = jnp.mean(x, axis=-1, keepdims=True)
    v = jnp.mean((x - m) * (x - m), axis=-1, keepdims=True)
    return (x - m) / jnp.sqrt(v + 1e-5) * g + b


def _prep_body(px_r, pn_r, pca_r, pc_r, pcb_r, dp_r, wg1_r,
               y_r, geom_r, dinv_r):
    deg = dp_r[0, :, 0:1] + dp_r[1, :, 0:1] + 1.0
    dinv = lax.rsqrt(deg)
    dinv_r[...] = dinv
    h = jnp.dot(px_r[...], wg1_r[...], preferred_element_type=_f32)
    y = h * dinv
    for q in range(4):
        y_r[q] = y[:, 32 * q:32 * q + 32]
    pn, pca, pc, pcb = pn_r[...], pca_r[...], pc_r[...], pcb_r[...]
    e0 = pn - pca
    e1 = pc - pca
    e0 = e0 / jnp.sqrt(jnp.sum(e0 * e0, axis=1, keepdims=True) + 1e-10)
    dot = jnp.sum(e0 * e1, axis=1, keepdims=True)
    e1 = e1 - e0 * dot
    e1 = e1 / jnp.sqrt(jnp.sum(e1 * e1, axis=1, keepdims=True) + 1e-10)
    a0, a1, a2 = e0[:, 0:1], e0[:, 1:2], e0[:, 2:3]
    b0, b1, b2 = e1[:, 0:1], e1[:, 1:2], e1[:, 2:3]
    e2 = jnp.concatenate(
        [a1 * b2 - a2 * b1, a2 * b0 - a0 * b2, a0 * b1 - a1 * b0], axis=1)
    zero = jnp.zeros((pn.shape[0], 1), _f32)
    geom_r[...] = jnp.concatenate([pcb, pca, e0, e1, e2, zero], axis=1)


def _prep_call(p_x, pos_n, pos_ca, pos_c, pos_cb, deg_parts, wg1):
    nb = 16
    blk = _N // nb
    return pl.pallas_call(
        _prep_body,
        grid=(nb,),
        in_specs=[
            pl.BlockSpec((blk, _DIN), lambda i: (i, 0)),
            pl.BlockSpec((blk, 3), lambda i: (i, 0)),
            pl.BlockSpec((blk, 3), lambda i: (i, 0)),
            pl.BlockSpec((blk, 3), lambda i: (i, 0)),
            pl.BlockSpec((blk, 3), lambda i: (i, 0)),
            pl.BlockSpec((2, blk, 16), lambda i: (0, i, 0)),
            pl.BlockSpec((_DIN, _DH), lambda i: (0, 0)),
        ],
        out_specs=[
            pl.BlockSpec((4, blk, 32), lambda i: (0, i, 0)),
            pl.BlockSpec((blk, 16), lambda i: (i, 0)),
            pl.BlockSpec((blk, 1), lambda i: (i, 0)),
        ],
        out_shape=[
            jax.ShapeDtypeStruct((4, _N, 32), _f32),
            jax.ShapeDtypeStruct((_N, 16), _f32),
            jax.ShapeDtypeStruct((_N, 1), _f32),
        ],
    )(p_x, pos_n, pos_ca, pos_c, pos_cb, deg_parts, wg1)


def _block_body(last, acc_r, dinv_r, geom_r, bg_r, lng_r, lnb_r,
                wq_r, wk_r, wv_r, wo_r, bo_r, ag_r, ab_r, wg2_r, out_r):
    acc = jnp.concatenate([acc_r[q] for q in range(4)], axis=1)
    dinv = dinv_r[...]
    x = _ln(jnp.maximum(dinv * acc + bg_r[...], 0.0), lng_r[...], lnb_r[...])
    geom = geom_r[...]
    geom_t = jnp.transpose(geom)
    q = jnp.dot(x, wq_r[...], preferred_element_type=_f32)
    k = jnp.dot(x, wk_r[...], preferred_element_type=_f32)
    v = jnp.dot(x, wv_r[...], preferred_element_type=_f32)
    fns, fps, dss, fds = [], [], [], []
    for h in range(_H):
        qh = q[:, h * _QK:(h + 1) * _QK]
        kh = k[:, h * _QK:(h + 1) * _QK]
        vh = v[:, h * _VD:(h + 1) * _VD]
        logits_t = lax.dot_general(kh, qh, (((1,), (1,)), ((), ())),
                                   preferred_element_type=_f32)
        m = jnp.max(logits_t, axis=0, keepdims=True)
        e = jnp.exp(logits_t - m)
        alpha_t = e / jnp.sum(e, axis=0, keepdims=True)
        vcat = jnp.concatenate([vh, geom], axis=1)
        u_t = lax.dot_general(vcat, alpha_t, (((0,), (0,)), ((), ())),
                              preferred_element_type=_f32)
        fns.append(u_t[:_VD])
        apb_t = u_t[_VD:_VD + 3] - geom_t[3:6]
        dist_t = jnp.sqrt(jnp.sum(apb_t * apb_t, axis=0, keepdims=True))
        fp_t = jnp.concatenate(
            [jnp.sum(geom_t[6 + 3 * i:9 + 3 * i] * apb_t, axis=0,
                     keepdims=True) for i in range(3)], axis=0)
        fd_t = fp_t / (jnp.sqrt(jnp.sum(fp_t * fp_t, axis=0, keepdims=True))
                       + 1e-10)
        fps.append(fp_t)
        dss.append(dist_t)
        fds.append(fd_t)
    pad = jnp.zeros((128 - _OD, _L), _f32)
    feat_t = jnp.concatenate(fns + fps + dss + fds + [pad], axis=0)
    fa = lax.dot_general(feat_t, wo_r[...], (((0,), (0,)), ((), ())),
                         preferred_element_type=_f32) + bo_r[...]
    px2 = x + _ln(x + fa, ag_r[...], ab_r[...])
    if last:
        out_r[...] = jnp.mean(px2, axis=0, keepdims=True)[None]
    else:
        y2 = jnp.dot(px2, wg2_r[...], preferred_element_type=_f32) * dinv
        for q in range(4):
            out_r[q] = y2[:, 32 * q:32 * q + 32]


def _block_call(last, acc, dinv, geom, bg, lng, lnb, wq, wk, wv, wo_pad, bo,
                ag, ab, wg2):
    if last:
        out_spec = pl.BlockSpec((1, 1, _DH), lambda b: (b, 0, 0))
        out_shape = jax.ShapeDtypeStruct((_B, 1, _DH), _f32)
    else:
        out_spec = pl.BlockSpec((4, _L, 32), lambda b: (0, b, 0))
        out_shape = jax.ShapeDtypeStruct((4, _N, 32), _f32)
    return pl.pallas_call(
        functools.partial(_block_body, last),
        grid=(_B,),
        in_specs=[
            pl.BlockSpec((4, _L, 32), lambda b: (0, b, 0)),
            pl.BlockSpec((_L, 1), lambda b: (b, 0)),
            pl.BlockSpec((_L, 16), lambda b: (b, 0)),
            pl.BlockSpec((1, _DH), lambda b: (0, 0)),
            pl.BlockSpec((1, _DH), lambda b: (0, 0)),
            pl.BlockSpec((1, _DH), lambda b: (0, 0)),
            pl.BlockSpec((_DH, _H * _QK), lambda b: (0, 0)),
            pl.BlockSpec((_DH, _H * _QK), lambda b: (0, 0)),
            pl.BlockSpec((_DH, _H * _VD), lambda b: (0, 0)),
            pl.BlockSpec((_DH, _DH), lambda b: (0, 0)),
            pl.BlockSpec((1, _DH), lambda b: (0, 0)),
            pl.BlockSpec((1, _DH), lambda b: (0, 0)),
            pl.BlockSpec((1, _DH), lambda b: (0, 0)),
            pl.BlockSpec((_DH, _DH), lambda b: (0, 0)),
        ],
        out_specs=out_spec,
        out_shape=out_shape,
    )(acc, dinv, geom, bg, lng, lnb, wq, wk, wv, wo_pad, bo, ag, ab, wg2)


def _head_body(gf_r, wf1_r, bf1_r, wf2_r, bf2_r, out_r):
    h = jnp.maximum(
        jnp.dot(gf_r[...], wf1_r[...], preferred_element_type=_f32)
        + bf1_r[...], 0.0)
    out_r[...] = (jnp.dot(h, wf2_r[...], preferred_element_type=_f32)
                  + bf2_r[...])


def _head_call(gf, wf1, bf1, wf2, bf2):
    return pl.pallas_call(
        _head_body,
        out_shape=jax.ShapeDtypeStruct((_B, _DH), _f32),
    )(gf, wf1, bf1, wf2, bf2)


def kernel(p_x, p_edge_index, pos_N, pos_CA, pos_C, pos_CB,
           Wg1, bg1, Wg2, bg2, ln1g, ln1b, ln2g, ln2b,
           Wq1, Wk1, Wv1, Wo1, bo1, ag1, ab1,
           Wq2, Wk2, Wv2, Wo2, bo2, ag2, ab2,
           Wf1, bf1, Wf2, bf2):
    src_r = p_edge_index[0].reshape(_ROWS, _CH)
    dst_r = p_edge_index[1].reshape(_ROWS, _CH)
    src_q = jnp.stack([src_r + q * _N for q in range(4)])

    deg_parts = _deg_call(dst_r).reshape(2, _N, 16)

    wo1p = jnp.concatenate([Wo1, jnp.zeros((128 - _OD, _DH), _f32)], axis=0)
    wo2p = jnp.concatenate([Wo2, jnp.zeros((128 - _OD, _DH), _f32)], axis=0)
    row = lambda t: t.reshape(1, -1)

    y1, geom, dinv = _prep_call(p_x, pos_N, pos_CA, pos_C, pos_CB,
                                deg_parts, Wg1)
    acc1 = _scat_call(y1.reshape(4 * _N, 32), src_q, dst_r)
    y2 = _block_call(False, acc1.reshape(4, _N, 32), dinv, geom,
                     row(bg1), row(ln1g), row(ln1b),
                     Wq1, Wk1, Wv1, wo1p, row(bo1), row(ag1), row(ab1), Wg2)
    acc2 = _scat_call(y2.reshape(4 * _N, 32), src_q, dst_r)
    gf = _block_call(True, acc2.reshape(4, _N, 32), dinv, geom,
                     row(bg2), row(ln2g), row(ln2b),
                     Wq2, Wk2, Wv2, wo2p, row(bo2), row(ag2), row(ab2), Wg2)
    return _head_call(gf.reshape(_B, _DH), Wf1, row(bf1), Wf2, row(bf2))
